# baseline (speedup 1.0000x reference)
import functools

import jax
import jax.numpy as jnp
from jax import lax
from jax.experimental import pallas as pl
from jax.experimental.pallas import tpu as pltpu
from jax.experimental.pallas import tpu_sc as plsc

B = 128
L = 8192
H = 64
V = 256
NUM_SLOTS = 128
BODY = L - 3

_NC = 2
_NS = 16
_NW = _NC * _NS
_RPW = B // _NW
_LANES = 16


_UNROLL = 16


def _sc_hist_body(seq_hbm, cnt_hbm, tail_hbm, seq_v, cnt_v, tail_v, sem):
    wid = lax.axis_index("s") * _NC + lax.axis_index("c")
    base = wid * _RPW
    ones = jnp.ones((_LANES,), jnp.int32)
    zeros = jnp.zeros((_LANES,), jnp.int32)
    nfull = BODY // _LANES
    rem = BODY - nfull * _LANES
    nun = nfull // _UNROLL
    ntail = nfull - nun * _UNROLL
    mask_rem = lax.iota(jnp.int32, _LANES) < rem

    copies = [pltpu.async_copy(seq_hbm.at[base], seq_v.at[0], sem), None]

    @plsc.parallel_loop(0, (_RPW * V) // _LANES, unroll=4)
    def zero_body(i):
        cnt_v[pl.ds(i * _LANES, _LANES)] = zeros

    for r in range(_RPW):
        if r + 1 < _RPW:
            copies[(r + 1) % 2] = pltpu.async_copy(
                seq_hbm.at[base + r + 1], seq_v.at[(r + 1) % 2], sem)
        copies[r % 2].wait()
        rb = r % 2
        row = cnt_v.at[pl.ds(r * V, V)]

        @plsc.parallel_loop(0, nun * _UNROLL, unroll=_UNROLL)
        def acc_body(i):
            tok = seq_v[rb, pl.ds(i * _LANES, _LANES)]
            plsc.addupdate_scatter(row, [tok], ones)

        for u in range(ntail):
            tok = seq_v[rb, pl.ds((nun * _UNROLL + u) * _LANES, _LANES)]
            plsc.addupdate_scatter(row, [tok], ones)
        if rem:
            tok = seq_v[rb, pl.ds(nfull * _LANES, _LANES)]
            plsc.addupdate_scatter(row, [tok], ones, mask=mask_rem)

        tail_v[r, :] = seq_v[rb, pl.ds(L - _LANES, _LANES)]

    pltpu.sync_copy(cnt_v, cnt_hbm.at[pl.ds(base * V, _RPW * V)])
    pltpu.sync_copy(tail_v, tail_hbm.at[pl.ds(base, _RPW)])


@functools.cache
def _sc_hist():
    return functools.partial(
        pl.kernel,
        out_type=(
            jax.ShapeDtypeStruct((B * V,), jnp.int32),
            jax.ShapeDtypeStruct((B, _LANES), jnp.int32),
        ),
        mesh=plsc.VectorSubcoreMesh(core_axis_name="c", subcore_axis_name="s"),
        compiler_params=pltpu.CompilerParams(needs_layout_passes=False),
        scratch_types=[
            pltpu.VMEM((2, L), jnp.int32),
            pltpu.VMEM((_RPW * V,), jnp.int32),
            pltpu.VMEM((_RPW, _LANES), jnp.int32),
            pltpu.SemaphoreType.DMA,
        ],
    )(_sc_hist_body)


def _mm(a, b, dims):
    return lax.dot_general(a, b, (dims, ((), ())),
                           precision=lax.Precision.HIGHEST)


def _tc_prep_body(embed_ref, W1_ref, b1_ref, W2_ref, b2_ref, gamma_ref,
                  beta_ref, ht_ref, higher_ref):
    f32 = jnp.float32

    e = embed_ref[...]
    ff = _mm(jnp.maximum(_mm(e, W1_ref[...], ((1,), (0,))) + b1_ref[...], 0.0),
             W2_ref[...], ((1,), (0,))) + b2_ref[...]
    x = e + ff
    m = jnp.mean(x, axis=1, keepdims=True)
    xc = x - m
    var = jnp.mean(xc * xc, axis=1, keepdims=True)
    ht = xc / jnp.sqrt(var + 1e-5) * gamma_ref[...] + beta_ref[...]
    ht_ref[...] = ht

    n2c = jnp.sum(ht * ht, axis=1, keepdims=True)
    n2r = _mm(jnp.ones((1, 1), f32), n2c, ((1,), (1,)))

    iu = lax.broadcasted_iota(jnp.int32, (V, V), 0)
    iv = lax.broadcasted_iota(jnp.int32, (V, V), 1)
    higher_ref[...] = jnp.where(
        (n2c > n2r) | ((n2c == n2r) & (iu < iv)), 1.0, 0.0)


_tc_prep = pl.pallas_call(
    _tc_prep_body,
    out_shape=(
        jax.ShapeDtypeStruct((V, H), jnp.float32),
        jax.ShapeDtypeStruct((V, V), jnp.float32),
    ),
)


def _tc_final_body(cnt_ref, tail_ref, ht_ref, higher_ref, Wq_ref, bq_ref,
                   Wo_ref, bo_ref, out_ref):
    ht = ht_ref[...]
    cnt = cnt_ref[...].astype(jnp.float32)
    ahead = _mm(cnt, higher_ref[...], ((1,), (0,)))
    m_sel = jnp.clip(jnp.float32(NUM_SLOTS) - ahead, 0.0, cnt)

    lt = tail_ref[...][:, _LANES - 1:_LANES]
    ohot = jnp.where(lt == lax.broadcasted_iota(jnp.int32, (B, V), 1), 1.0, 0.0)
    hlast = _mm(ohot, ht, ((1,), (0,)))
    q = _mm(hlast, Wq_ref[...], ((1,), (0,))) + bq_ref[...]

    s = _mm(q, ht, ((1,), (1,))) * jnp.float32(1.0 / (H ** 0.5))
    sel = m_sel > 0.0
    smax = jnp.max(jnp.where(sel, s, -1e30), axis=1, keepdims=True)
    w = jnp.where(sel, m_sel * jnp.exp(s - smax), 0.0)
    den = jnp.sum(w, axis=1, keepdims=True)
    pooled = _mm(w, ht, ((1,), (0,))) / den
    out_ref[...] = _mm(pooled, Wo_ref[...], ((1,), (0,))) + bo_ref[...]


_tc_final = pl.pallas_call(
    _tc_final_body,
    out_shape=jax.ShapeDtypeStruct((B, V), jnp.float32),
)


def kernel(seq, embed, W1, b1, W2, b2, gamma, beta, Wq, bq, Wo, bo):
    cnt, tail = _sc_hist()(seq)
    cnt = cnt.reshape(B, V)
    ht, higher = _tc_prep(
        embed, W1, b1.reshape(1, -1), W2, b2.reshape(1, -1),
        gamma.reshape(1, -1), beta.reshape(1, -1))
    return _tc_final(cnt, tail, ht, higher, Wq, bq.reshape(1, -1),
                     Wo, bo.reshape(1, -1))


# speedup vs baseline: 1.0203x; 1.0203x over previous
import functools

import jax
import jax.numpy as jnp
from jax import lax
from jax.experimental import pallas as pl
from jax.experimental.pallas import tpu as pltpu
from jax.experimental.pallas import tpu_sc as plsc

B = 128
L = 8192
H = 64
V = 256
NUM_SLOTS = 128
BODY = L - 3

_NC = 2
_NS = 16
_NW = _NC * _NS
_RPW = B // _NW
_LANES = 16


_UNROLL = 16


def _sc_hist_body(seq_hbm, cnt_hbm, tail_hbm, seq_v, cnt_v, tail_v, sem):
    wid = lax.axis_index("s") * _NC + lax.axis_index("c")
    base = wid * _RPW
    ones = jnp.ones((_LANES,), jnp.int32)
    zeros = jnp.zeros((_LANES,), jnp.int32)
    nfull = BODY // _LANES
    rem = BODY - nfull * _LANES
    nun = nfull // _UNROLL
    ntail = nfull - nun * _UNROLL
    mask_rem = lax.iota(jnp.int32, _LANES) < rem

    copies = [pltpu.async_copy(seq_hbm.at[base + r], seq_v.at[r], sem)
              for r in range(_RPW)]

    @plsc.parallel_loop(0, (_RPW * V) // _LANES, unroll=4)
    def zero_body(i):
        cnt_v[pl.ds(i * _LANES, _LANES)] = zeros

    for r in range(_RPW):
        copies[r].wait()
        rb = r
        row = cnt_v.at[pl.ds(r * V, V)]

        @plsc.parallel_loop(0, nun * _UNROLL, unroll=_UNROLL)
        def acc_body(i):
            tok = seq_v[rb, pl.ds(i * _LANES, _LANES)]
            plsc.addupdate_scatter(row, [tok], ones)

        for u in range(ntail):
            tok = seq_v[rb, pl.ds((nun * _UNROLL + u) * _LANES, _LANES)]
            plsc.addupdate_scatter(row, [tok], ones)
        if rem:
            tok = seq_v[rb, pl.ds(nfull * _LANES, _LANES)]
            plsc.addupdate_scatter(row, [tok], ones, mask=mask_rem)

        tail_v[r, :] = seq_v[rb, pl.ds(L - _LANES, _LANES)]

    pltpu.sync_copy(cnt_v, cnt_hbm.at[pl.ds(base * V, _RPW * V)])
    pltpu.sync_copy(tail_v, tail_hbm.at[pl.ds(base, _RPW)])


@functools.cache
def _sc_hist():
    return functools.partial(
        pl.kernel,
        out_type=(
            jax.ShapeDtypeStruct((B * V,), jnp.int32),
            jax.ShapeDtypeStruct((B, _LANES), jnp.int32),
        ),
        mesh=plsc.VectorSubcoreMesh(core_axis_name="c", subcore_axis_name="s"),
        compiler_params=pltpu.CompilerParams(needs_layout_passes=False),
        scratch_types=[
            pltpu.VMEM((_RPW, L), jnp.int32),
            pltpu.VMEM((_RPW * V,), jnp.int32),
            pltpu.VMEM((_RPW, _LANES), jnp.int32),
            pltpu.SemaphoreType.DMA,
        ],
    )(_sc_hist_body)


def _mm(a, b, dims):
    return lax.dot_general(a, b, (dims, ((), ())),
                           precision=lax.Precision.HIGHEST)


def _tc_prep_body(embed_ref, W1_ref, b1_ref, W2_ref, b2_ref, gamma_ref,
                  beta_ref, ht_ref, higher_ref):
    f32 = jnp.float32

    e = embed_ref[...]
    ff = _mm(jnp.maximum(_mm(e, W1_ref[...], ((1,), (0,))) + b1_ref[...], 0.0),
             W2_ref[...], ((1,), (0,))) + b2_ref[...]
    x = e + ff
    m = jnp.mean(x, axis=1, keepdims=True)
    xc = x - m
    var = jnp.mean(xc * xc, axis=1, keepdims=True)
    ht = xc / jnp.sqrt(var + 1e-5) * gamma_ref[...] + beta_ref[...]
    ht_ref[...] = ht

    n2c = jnp.sum(ht * ht, axis=1, keepdims=True)
    n2r = _mm(jnp.ones((1, 1), f32), n2c, ((1,), (1,)))

    iu = lax.broadcasted_iota(jnp.int32, (V, V), 0)
    iv = lax.broadcasted_iota(jnp.int32, (V, V), 1)
    higher_ref[...] = jnp.where(
        (n2c > n2r) | ((n2c == n2r) & (iu < iv)), 1.0, 0.0)


_tc_prep = pl.pallas_call(
    _tc_prep_body,
    out_shape=(
        jax.ShapeDtypeStruct((V, H), jnp.float32),
        jax.ShapeDtypeStruct((V, V), jnp.float32),
    ),
)


def _tc_final_body(cnt_ref, tail_ref, ht_ref, higher_ref, Wq_ref, bq_ref,
                   Wo_ref, bo_ref, out_ref):
    ht = ht_ref[...]
    cnt = cnt_ref[...].astype(jnp.float32)
    ahead = _mm(cnt, higher_ref[...], ((1,), (0,)))
    m_sel = jnp.clip(jnp.float32(NUM_SLOTS) - ahead, 0.0, cnt)

    lt = tail_ref[...][:, _LANES - 1:_LANES]
    ohot = jnp.where(lt == lax.broadcasted_iota(jnp.int32, (B, V), 1), 1.0, 0.0)
    hlast = _mm(ohot, ht, ((1,), (0,)))
    q = _mm(hlast, Wq_ref[...], ((1,), (0,))) + bq_ref[...]

    s = _mm(q, ht, ((1,), (1,))) * jnp.float32(1.0 / (H ** 0.5))
    sel = m_sel > 0.0
    smax = jnp.max(jnp.where(sel, s, -1e30), axis=1, keepdims=True)
    w = jnp.where(sel, m_sel * jnp.exp(s - smax), 0.0)
    den = jnp.sum(w, axis=1, keepdims=True)
    pooled = _mm(w, ht, ((1,), (0,))) / den
    out_ref[...] = _mm(pooled, Wo_ref[...], ((1,), (0,))) + bo_ref[...]


_tc_final = pl.pallas_call(
    _tc_final_body,
    out_shape=jax.ShapeDtypeStruct((B, V), jnp.float32),
)


def kernel(seq, embed, W1, b1, W2, b2, gamma, beta, Wq, bq, Wo, bo):
    cnt, tail = _sc_hist()(seq)
    cnt = cnt.reshape(B, V)
    ht, higher = _tc_prep(
        embed, W1, b1.reshape(1, -1), W2, b2.reshape(1, -1),
        gamma.reshape(1, -1), beta.reshape(1, -1))
    return _tc_final(cnt, tail, ht, higher, Wq, bq.reshape(1, -1),
                     Wo, bo.reshape(1, -1))


# speedup vs baseline: 1.0465x; 1.0258x over previous
import functools

import jax
import jax.numpy as jnp
from jax import lax
from jax.experimental import pallas as pl
from jax.experimental.pallas import tpu as pltpu
from jax.experimental.pallas import tpu_sc as plsc

B = 128
L = 8192
H = 64
V = 256
NUM_SLOTS = 128
BODY = L - 3

_NC = 2
_NS = 16
_NW = _NC * _NS
_RPW = B // _NW
_LANES = 16


_UNROLL = 16


def _sc_hist_body(seq_hbm, cnt_hbm, seq_v, cnt_v, sem):
    wid = lax.axis_index("s") * _NC + lax.axis_index("c")
    base = wid * _RPW
    ones = jnp.ones((_LANES,), jnp.int32)
    zeros = jnp.zeros((_LANES,), jnp.int32)
    nfull = BODY // _LANES
    rem = BODY - nfull * _LANES
    nun = nfull // _UNROLL
    ntail = nfull - nun * _UNROLL
    mask_rem = lax.iota(jnp.int32, _LANES) < rem

    copies = [pltpu.async_copy(seq_hbm.at[base + r], seq_v.at[r], sem)
              for r in range(_RPW)]

    @plsc.parallel_loop(0, (_RPW * V) // _LANES, unroll=4)
    def zero_body(i):
        cnt_v[pl.ds(i * _LANES, _LANES)] = zeros

    for r in range(_RPW):
        copies[r].wait()
        rb = r
        row = cnt_v.at[pl.ds(r * V, V)]

        @plsc.parallel_loop(0, nun * _UNROLL, unroll=_UNROLL)
        def acc_body(i):
            tok = seq_v[rb, pl.ds(i * _LANES, _LANES)]
            plsc.addupdate_scatter(row, [tok], ones)

        for u in range(ntail):
            tok = seq_v[rb, pl.ds((nun * _UNROLL + u) * _LANES, _LANES)]
            plsc.addupdate_scatter(row, [tok], ones)
        if rem:
            tok = seq_v[rb, pl.ds(nfull * _LANES, _LANES)]
            plsc.addupdate_scatter(row, [tok], ones, mask=mask_rem)

    pltpu.sync_copy(cnt_v, cnt_hbm.at[pl.ds(base * V, _RPW * V)])


@functools.cache
def _sc_hist():
    return functools.partial(
        pl.kernel,
        out_type=jax.ShapeDtypeStruct((B * V,), jnp.int32),
        mesh=plsc.VectorSubcoreMesh(core_axis_name="c", subcore_axis_name="s"),
        compiler_params=pltpu.CompilerParams(needs_layout_passes=False),
        scratch_types=[
            pltpu.VMEM((_RPW, L), jnp.int32),
            pltpu.VMEM((_RPW * V,), jnp.int32),
            pltpu.SemaphoreType.DMA,
        ],
    )(_sc_hist_body)


def _mm(a, b, dims):
    return lax.dot_general(a, b, (dims, ((), ())),
                           precision=lax.Precision.HIGHEST)


def _tc_prep_body(last_ref, embed_ref, W1_ref, b1_ref, W2_ref, b2_ref,
                  gamma_ref, beta_ref, Wq_ref, bq_ref, ht_ref, higher_ref,
                  ew_ref):
    f32 = jnp.float32

    e = embed_ref[...]
    ff = _mm(jnp.maximum(_mm(e, W1_ref[...], ((1,), (0,))) + b1_ref[...], 0.0),
             W2_ref[...], ((1,), (0,))) + b2_ref[...]
    x = e + ff
    m = jnp.mean(x, axis=1, keepdims=True)
    xc = x - m
    var = jnp.mean(xc * xc, axis=1, keepdims=True)
    ht = xc / jnp.sqrt(var + 1e-5) * gamma_ref[...] + beta_ref[...]
    ht_ref[...] = ht

    n2c = jnp.sum(ht * ht, axis=1, keepdims=True)
    n2r = _mm(jnp.ones((1, 1), f32), n2c, ((1,), (1,)))

    iu = lax.broadcasted_iota(jnp.int32, (V, V), 0)
    iv = lax.broadcasted_iota(jnp.int32, (V, V), 1)
    higher_ref[...] = jnp.where(
        (n2c > n2r) | ((n2c == n2r) & (iu < iv)), 1.0, 0.0)

    lt = last_ref[...]
    ohot = jnp.where(lt == lax.broadcasted_iota(jnp.int32, (B, V), 1), 1.0, 0.0)
    hlast = _mm(ohot, ht, ((1,), (0,)))
    q = _mm(hlast, Wq_ref[...], ((1,), (0,))) + bq_ref[...]
    s = _mm(q, ht, ((1,), (1,))) * jnp.float32(1.0 / (H ** 0.5))
    smax = jnp.max(s, axis=1, keepdims=True)
    ew_ref[...] = jnp.exp(s - smax)


_tc_prep = pl.pallas_call(
    _tc_prep_body,
    out_shape=(
        jax.ShapeDtypeStruct((V, H), jnp.float32),
        jax.ShapeDtypeStruct((V, V), jnp.float32),
        jax.ShapeDtypeStruct((B, V), jnp.float32),
    ),
)


def _tc_final_body(cnt_ref, ht_ref, higher_ref, ew_ref, Wo_ref, bo_ref,
                   out_ref):
    cnt = cnt_ref[...].astype(jnp.float32)
    ahead = _mm(cnt, higher_ref[...], ((1,), (0,)))
    m_sel = jnp.clip(jnp.float32(NUM_SLOTS) - ahead, 0.0, cnt)

    w = m_sel * ew_ref[...]
    den = jnp.sum(w, axis=1, keepdims=True)
    pooled = _mm(w, ht_ref[...], ((1,), (0,))) / den
    out_ref[...] = _mm(pooled, Wo_ref[...], ((1,), (0,))) + bo_ref[...]


_tc_final = pl.pallas_call(
    _tc_final_body,
    out_shape=jax.ShapeDtypeStruct((B, V), jnp.float32),
)


def kernel(seq, embed, W1, b1, W2, b2, gamma, beta, Wq, bq, Wo, bo):
    cnt = _sc_hist()(seq).reshape(B, V)
    last = seq[:, L - 1:L]
    ht, higher, ew = _tc_prep(
        last, embed, W1, b1.reshape(1, -1), W2, b2.reshape(1, -1),
        gamma.reshape(1, -1), beta.reshape(1, -1), Wq, bq.reshape(1, -1))
    return _tc_final(cnt, ht, higher, ew, Wo, bo.reshape(1, -1))
